# baseline (speedup 1.0000x reference)
import functools

import jax
import jax.numpy as jnp
from jax.experimental import pallas as pl
from jax.experimental.pallas import tpu as pltpu

N = 512
D = 128
T = 4
GH = 64
NE = 64
SE = 128
NC = 10

_F32 = jnp.float32


def _mm(a, b, dims):
    return jax.lax.dot_general(
        a, b, dimension_numbers=(dims, ((), ())),
        preferred_element_type=_F32)


def _mmb(a, b, dims):
    return jax.lax.dot_general(
        a.astype(jnp.bfloat16), b.astype(jnp.bfloat16),
        dimension_numbers=(dims, ((), ())),
        preferred_element_type=_F32)


def _body(adj_ref, x_ref, xT_ref, W1_ref, b1_ref, W2_ref, b2_ref,
          W_ih1_ref, W_hh1_ref, bg1_ref, Wsw_ref, bsw_ref,
          W_ih2_ref, W_hh2_ref, bg2_ref, Wc_ref, bc_ref, out_ref,
          gin_scr):
    W1 = W1_ref[...]
    W2 = W2_ref[...]
    W_hh1 = W_hh1_ref[...]

    rows = jax.lax.broadcasted_iota(jnp.int32, (N, N), 0)
    cols = jax.lax.broadcasted_iota(jnp.int32, (N, N), 1)
    eye = rows == cols
    ones_col = jnp.ones((N, 1), _F32)

    gvs = []
    for t in range(T):
        adj = adj_ref[t]
        x = x_ref[t]

        Aw = jnp.where(adj > 0, adj, 0.0)
        d_col = jnp.sum(jnp.where(eye, Aw, 0.0), axis=1, keepdims=True)
        Aw = Aw + jnp.where(eye, jnp.where(d_col > 0, 0.0, 1.0), 0.0)
        deg_col = _mm(Aw, ones_col, ((0,), (0,)))
        dinv = jnp.where(deg_col > 0, jax.lax.rsqrt(deg_col), 0.0)

        Awb = Aw.astype(jnp.bfloat16)

        def prop(y, Awb=Awb, dinv=dinv):
            return dinv * _mmb(Awb, dinv * y, ((0,), (0,)))

        h1 = jax.nn.relu(prop(_mmb(x, W1, ((1,), (0,)))) + b1_ref[...])
        h2 = jax.nn.relu(prop(_mmb(h1, W2, ((1,), (0,)))) + b2_ref[...])
        gvs.append(jnp.mean(h2, axis=0, keepdims=True))

    gin_scr[...] = _mmb(xT_ref[...], W_ih1_ref[...], ((1,), (1,))) + bg1_ref[...]

    W_hh1b = W_hh1.astype(jnp.bfloat16)

    def substep(gin, H, C):
        g = gin + _mmb(H, W_hh1b, ((1,), (1,)))
        i = jax.nn.sigmoid(g[:, :SE])
        f = jax.nn.sigmoid(g[:, SE:2 * SE])
        gg = jnp.tanh(g[:, 2 * SE:3 * SE])
        o = jax.nn.sigmoid(g[:, 3 * SE:])
        C = f * C + i * gg
        return o * jnp.tanh(C), C

    def step(s, carry):
        H, C = carry
        gin8 = gin_scr[pl.ds(s * 2 * T, 2 * T), :]
        H, C = substep(gin8[:T], H, C)
        H, C = substep(gin8[T:], H, C)
        return H, C

    H0 = jnp.zeros((T, SE), _F32)
    H1, _ = jax.lax.fori_loop(0, D // 2, step, (H0, H0), unroll=4)

    sg = jnp.concatenate([H1, jnp.concatenate(gvs, axis=0)], axis=1)
    sv = jax.nn.relu(_mm(sg, Wsw_ref[...], ((1,), (0,))) + bsw_ref[...])

    W_hh2 = W_hh2_ref[...]
    gin2 = _mm(sv, W_ih2_ref[...], ((1,), (1,))) + bg2_ref[...]
    h = jnp.zeros((1, SE), _F32)
    c = jnp.zeros((1, SE), _F32)
    for s in range(T):
        g = gin2[s:s + 1] + _mm(h, W_hh2, ((1,), (1,)))
        i = jax.nn.sigmoid(g[:, :SE])
        f = jax.nn.sigmoid(g[:, SE:2 * SE])
        gg = jnp.tanh(g[:, 2 * SE:3 * SE])
        o = jax.nn.sigmoid(g[:, 3 * SE:])
        c = f * c + i * gg
        h = o * jnp.tanh(c)

    logits = _mm(h, Wc_ref[...], ((1,), (0,))) + bc_ref[...]
    m = jnp.max(logits, axis=1, keepdims=True)
    z = logits - m
    out_ref[...] = z - jnp.log(jnp.sum(jnp.exp(z), axis=1, keepdims=True))


def _forward(adj_mat_array, node_att_array, W1, b1, W2, b2, W_ih1, W_hh1,
             b_ih1, b_hh1, Wsw, bsw, W_ih2, W_hh2, b_ih2, b_hh2, Wc, bc,
             interpret=False):
    bg1 = (b_ih1 + b_hh1).reshape(1, -1)
    bg2 = (b_ih2 + b_hh2).reshape(1, -1)
    xT = node_att_array.transpose(2, 0, 1).reshape(D * T, N)
    return pl.pallas_call(
        _body,
        out_shape=jax.ShapeDtypeStruct((1, NC), _F32),
        scratch_shapes=[pltpu.VMEM((D * T, 4 * SE), _F32)],
        interpret=interpret,
    )(adj_mat_array, node_att_array, xT, W1, b1.reshape(1, -1), W2,
      b2.reshape(1, -1), W_ih1, W_hh1, bg1, Wsw, bsw.reshape(1, -1),
      W_ih2, W_hh2, bg2, Wc, bc.reshape(1, -1))


def kernel(adj_mat_array, node_att_array, W1, b1, W2, b2, W_ih1, W_hh1,
           b_ih1, b_hh1, Wsw, bsw, W_ih2, W_hh2, b_ih2, b_hh2, Wc, bc):
    return _forward(adj_mat_array, node_att_array, W1, b1, W2, b2, W_ih1,
                    W_hh1, b_ih1, b_hh1, Wsw, bsw, W_ih2, W_hh2, b_ih2,
                    b_hh2, Wc, bc)


# speedup vs baseline: 1.0213x; 1.0213x over previous
import functools

import jax
import jax.numpy as jnp
from jax.experimental import pallas as pl
from jax.experimental.pallas import tpu as pltpu

N = 512
D = 128
T = 4
GH = 64
NE = 64
SE = 128
NC = 10

_F32 = jnp.float32


def _mm(a, b, dims):
    return jax.lax.dot_general(
        a, b, dimension_numbers=(dims, ((), ())),
        preferred_element_type=_F32)


def _mmb(a, b, dims):
    return jax.lax.dot_general(
        a.astype(jnp.bfloat16), b.astype(jnp.bfloat16),
        dimension_numbers=(dims, ((), ())),
        preferred_element_type=_F32)


def _body(adj_ref, x_ref, xT_ref, W1_ref, b1_ref, W2_ref, b2_ref,
          W_ih1_ref, W_hh1_ref, bg1_ref, Wsw_ref, bsw_ref,
          W_ih2_ref, W_hh2_ref, bg2_ref, Wc_ref, bc_ref, out_ref,
          gin_scr):
    W1 = W1_ref[...]
    W2 = W2_ref[...]
    W_hh1 = W_hh1_ref[...]

    rows = jax.lax.broadcasted_iota(jnp.int32, (N, N), 0)
    cols = jax.lax.broadcasted_iota(jnp.int32, (N, N), 1)
    eye = rows == cols
    ones_col = jnp.ones((N, 1), _F32)

    gvs = []
    for t in range(T):
        adj = adj_ref[t]
        x = x_ref[t]

        Aw = jnp.where(adj > 0, adj, 0.0)
        d_col = jnp.sum(jnp.where(eye, Aw, 0.0), axis=1, keepdims=True)
        Aw = Aw + jnp.where(eye, jnp.where(d_col > 0, 0.0, 1.0), 0.0)
        deg_col = _mm(Aw, ones_col, ((0,), (0,)))
        dinv = jnp.where(deg_col > 0, jax.lax.rsqrt(deg_col), 0.0)

        Awb = Aw.astype(jnp.bfloat16)

        def prop(y, Awb=Awb, dinv=dinv):
            return dinv * _mmb(Awb, dinv * y, ((0,), (0,)))

        h1 = jax.nn.relu(prop(_mmb(x, W1, ((1,), (0,)))) + b1_ref[...])
        h2 = jax.nn.relu(prop(_mmb(h1, W2, ((1,), (0,)))) + b2_ref[...])
        gvs.append(jnp.mean(h2, axis=0, keepdims=True))

    gin_scr[...] = _mmb(xT_ref[...], W_ih1_ref[...], ((1,), (1,))) + bg1_ref[...]

    W_hh1b = W_hh1.astype(jnp.bfloat16)

    def substep(gin, H, C):
        g = gin + _mmb(H, W_hh1b, ((1,), (1,)))
        i = jax.nn.sigmoid(g[:, :SE])
        f = jax.nn.sigmoid(g[:, SE:2 * SE])
        gg = jnp.tanh(g[:, 2 * SE:3 * SE])
        o = jax.nn.sigmoid(g[:, 3 * SE:])
        C = f * C + i * gg
        return o * jnp.tanh(C), C

    H = jnp.zeros((T, SE), _F32)
    C = jnp.zeros((T, SE), _F32)
    for s in range(D // 2):
        gin8 = gin_scr[s * 2 * T:(s + 1) * 2 * T, :]
        H, C = substep(gin8[:T], H, C)
        H, C = substep(gin8[T:], H, C)
    H1 = H

    sg = jnp.concatenate([H1, jnp.concatenate(gvs, axis=0)], axis=1)
    sv = jax.nn.relu(_mm(sg, Wsw_ref[...], ((1,), (0,))) + bsw_ref[...])

    W_hh2 = W_hh2_ref[...]
    gin2 = _mm(sv, W_ih2_ref[...], ((1,), (1,))) + bg2_ref[...]
    h = jnp.zeros((1, SE), _F32)
    c = jnp.zeros((1, SE), _F32)
    for s in range(T):
        g = gin2[s:s + 1] + _mm(h, W_hh2, ((1,), (1,)))
        i = jax.nn.sigmoid(g[:, :SE])
        f = jax.nn.sigmoid(g[:, SE:2 * SE])
        gg = jnp.tanh(g[:, 2 * SE:3 * SE])
        o = jax.nn.sigmoid(g[:, 3 * SE:])
        c = f * c + i * gg
        h = o * jnp.tanh(c)

    logits = _mm(h, Wc_ref[...], ((1,), (0,))) + bc_ref[...]
    m = jnp.max(logits, axis=1, keepdims=True)
    z = logits - m
    out_ref[...] = z - jnp.log(jnp.sum(jnp.exp(z), axis=1, keepdims=True))


def _forward(adj_mat_array, node_att_array, W1, b1, W2, b2, W_ih1, W_hh1,
             b_ih1, b_hh1, Wsw, bsw, W_ih2, W_hh2, b_ih2, b_hh2, Wc, bc,
             interpret=False):
    bg1 = (b_ih1 + b_hh1).reshape(1, -1)
    bg2 = (b_ih2 + b_hh2).reshape(1, -1)
    xT = node_att_array.transpose(2, 0, 1).reshape(D * T, N)
    return pl.pallas_call(
        _body,
        out_shape=jax.ShapeDtypeStruct((1, NC), _F32),
        scratch_shapes=[pltpu.VMEM((D * T, 4 * SE), _F32)],
        interpret=interpret,
    )(adj_mat_array, node_att_array, xT, W1, b1.reshape(1, -1), W2,
      b2.reshape(1, -1), W_ih1, W_hh1, bg1, Wsw, bsw.reshape(1, -1),
      W_ih2, W_hh2, bg2, Wc, bc.reshape(1, -1))


def kernel(adj_mat_array, node_att_array, W1, b1, W2, b2, W_ih1, W_hh1,
           b_ih1, b_hh1, Wsw, bsw, W_ih2, W_hh2, b_ih2, b_hh2, Wc, bc):
    return _forward(adj_mat_array, node_att_array, W1, b1, W2, b2, W_ih1,
                    W_hh1, b_ih1, b_hh1, Wsw, bsw, W_ih2, W_hh2, b_ih2,
                    b_hh2, Wc, bc)


# speedup vs baseline: 1.0533x; 1.0314x over previous
import functools

import jax
import jax.numpy as jnp
from jax.experimental import pallas as pl
from jax.experimental.pallas import tpu as pltpu

N = 512
D = 128
T = 4
GH = 64
NE = 64
SE = 128
NC = 10

_F32 = jnp.float32


def _mm(a, b, dims):
    return jax.lax.dot_general(
        a, b, dimension_numbers=(dims, ((), ())),
        preferred_element_type=_F32)


def _mmb(a, b, dims):
    return jax.lax.dot_general(
        a.astype(jnp.bfloat16), b.astype(jnp.bfloat16),
        dimension_numbers=(dims, ((), ())),
        preferred_element_type=_F32)


def _sig(x):
    return 0.5 * jnp.tanh(0.5 * x) + 0.5


def _body(adj_ref, x_ref, W1_ref, b1_ref, W2_ref, b2_ref,
          W_ih1_ref, W_hh1_ref, bg1_ref, Wsw_ref, bsw_ref,
          W_ih2_ref, W_hh2_ref, bg2_ref, Wc_ref, bc_ref, out_ref,
          *gin_scr):
    W1 = W1_ref[...]
    W2 = W2_ref[...]
    W_hh1 = W_hh1_ref[...]

    rows = jax.lax.broadcasted_iota(jnp.int32, (N, N), 0)
    cols = jax.lax.broadcasted_iota(jnp.int32, (N, N), 1)
    eye = rows == cols
    ones_col = jnp.ones((N, 1), _F32)

    gvs = []
    for t in range(T):
        adj = adj_ref[t]
        x = x_ref[t]

        Aw = jnp.where(adj > 0, adj, 0.0)
        d_col = jnp.sum(jnp.where(eye, Aw, 0.0), axis=1, keepdims=True)
        Aw = Aw + jnp.where(eye, jnp.where(d_col > 0, 0.0, 1.0), 0.0)
        deg_col = _mm(Aw, ones_col, ((0,), (0,)))
        dinv = jnp.where(deg_col > 0, jax.lax.rsqrt(deg_col), 0.0)

        Awb = Aw.astype(jnp.bfloat16)

        def prop(y, Awb=Awb, dinv=dinv):
            return dinv * _mmb(Awb, dinv * y, ((0,), (0,)))

        h1 = jax.nn.relu(prop(_mmb(x, W1, ((1,), (0,)))) + b1_ref[...])
        h2 = jax.nn.relu(prop(_mmb(h1, W2, ((1,), (0,)))) + b2_ref[...])
        gvs.append(jnp.mean(h2, axis=0, keepdims=True))

        gin_scr[t][...] = _mmb(x, W_ih1_ref[...], ((0,), (1,))) + bg1_ref[...]

    W_hh1b = W_hh1.astype(jnp.bfloat16)

    def substep(gin, H, C):
        g = gin + _mmb(H, W_hh1b, ((1,), (1,)))
        i = _sig(g[:, :SE])
        f = _sig(g[:, SE:2 * SE])
        gg = jnp.tanh(g[:, 2 * SE:3 * SE])
        o = _sig(g[:, 3 * SE:])
        C = f * C + i * gg
        return o * jnp.tanh(C), C

    H = jnp.zeros((T, SE), _F32)
    C = jnp.zeros((T, SE), _F32)
    for s in range(D):
        gin = jnp.concatenate([g[s:s + 1, :] for g in gin_scr], axis=0)
        H, C = substep(gin, H, C)
    H1 = H

    sg = jnp.concatenate([H1, jnp.concatenate(gvs, axis=0)], axis=1)
    sv = jax.nn.relu(_mm(sg, Wsw_ref[...], ((1,), (0,))) + bsw_ref[...])

    W_hh2 = W_hh2_ref[...]
    gin2 = _mm(sv, W_ih2_ref[...], ((1,), (1,))) + bg2_ref[...]
    h = jnp.zeros((1, SE), _F32)
    c = jnp.zeros((1, SE), _F32)
    for s in range(T):
        g = gin2[s:s + 1] + _mm(h, W_hh2, ((1,), (1,)))
        i = _sig(g[:, :SE])
        f = _sig(g[:, SE:2 * SE])
        gg = jnp.tanh(g[:, 2 * SE:3 * SE])
        o = _sig(g[:, 3 * SE:])
        c = f * c + i * gg
        h = o * jnp.tanh(c)

    logits = _mm(h, Wc_ref[...], ((1,), (0,))) + bc_ref[...]
    m = jnp.max(logits, axis=1, keepdims=True)
    z = logits - m
    out_ref[...] = z - jnp.log(jnp.sum(jnp.exp(z), axis=1, keepdims=True))


def _forward(adj_mat_array, node_att_array, W1, b1, W2, b2, W_ih1, W_hh1,
             b_ih1, b_hh1, Wsw, bsw, W_ih2, W_hh2, b_ih2, b_hh2, Wc, bc,
             interpret=False):
    bg1 = (b_ih1 + b_hh1).reshape(1, -1)
    bg2 = (b_ih2 + b_hh2).reshape(1, -1)
    return pl.pallas_call(
        _body,
        out_shape=jax.ShapeDtypeStruct((1, NC), _F32),
        scratch_shapes=[pltpu.VMEM((D, 4 * SE), _F32) for _ in range(T)],
        interpret=interpret,
    )(adj_mat_array, node_att_array, W1, b1.reshape(1, -1), W2,
      b2.reshape(1, -1), W_ih1, W_hh1, bg1, Wsw, bsw.reshape(1, -1),
      W_ih2, W_hh2, bg2, Wc, bc.reshape(1, -1))


def kernel(adj_mat_array, node_att_array, W1, b1, W2, b2, W_ih1, W_hh1,
           b_ih1, b_hh1, Wsw, bsw, W_ih2, W_hh2, b_ih2, b_hh2, Wc, bc):
    return _forward(adj_mat_array, node_att_array, W1, b1, W2, b2, W_ih1,
                    W_hh1, b_ih1, b_hh1, Wsw, bsw, W_ih2, W_hh2, b_ih2,
                    b_hh2, Wc, bc)


# speedup vs baseline: 1.0586x; 1.0050x over previous
import functools

import jax
import jax.numpy as jnp
from jax.experimental import pallas as pl
from jax.experimental.pallas import tpu as pltpu

N = 512
D = 128
T = 4
GH = 64
NE = 64
SE = 128
NC = 10

_F32 = jnp.float32


def _mm(a, b, dims):
    return jax.lax.dot_general(
        a, b, dimension_numbers=(dims, ((), ())),
        preferred_element_type=_F32)


def _mmb(a, b, dims):
    return jax.lax.dot_general(
        a.astype(jnp.bfloat16), b.astype(jnp.bfloat16),
        dimension_numbers=(dims, ((), ())),
        preferred_element_type=_F32)


def _sig(x):
    return 0.5 * jnp.tanh(0.5 * x) + 0.5


def _body(adj_ref, x_ref, W1_ref, b1_ref, W2_ref, b2_ref,
          W_ih1_ref, W_hh1_ref, bg1_ref, Wsw_ref, bsw_ref,
          W_ih2_ref, W_hh2_ref, bg2_ref, Wc_ref, bc_ref, out_ref,
          *gin_scr):
    W1 = W1_ref[...]
    W2 = W2_ref[...]
    W_hh1 = W_hh1_ref[...]

    rows = jax.lax.broadcasted_iota(jnp.int32, (N, N), 0)
    cols = jax.lax.broadcasted_iota(jnp.int32, (N, N), 1)
    eye = rows == cols
    ones_col = jnp.ones((N, 1), _F32)

    for t in range(T):
        gin_scr[t][...] = (_mmb(x_ref[t], W_ih1_ref[...], ((0,), (1,)))
                           + bg1_ref[...])

    def gcn(t):
        adj = adj_ref[t]
        x = x_ref[t]

        Aw = jnp.where(adj > 0, adj, 0.0)
        d_col = jnp.sum(jnp.where(eye, Aw, 0.0), axis=1, keepdims=True)
        Aw = Aw + jnp.where(eye, jnp.where(d_col > 0, 0.0, 1.0), 0.0)
        deg_col = _mm(Aw, ones_col, ((0,), (0,)))
        dinv = jnp.where(deg_col > 0, jax.lax.rsqrt(deg_col), 0.0)

        Awb = Aw.astype(jnp.bfloat16)

        def prop(y):
            return dinv * _mmb(Awb, dinv * y, ((0,), (0,)))

        h1 = jax.nn.relu(prop(_mmb(x, W1, ((1,), (0,)))) + b1_ref[...])
        h2 = jax.nn.relu(prop(_mmb(h1, W2, ((1,), (0,)))) + b2_ref[...])
        return jnp.mean(h2, axis=0, keepdims=True)

    W_hh1b = W_hh1.astype(jnp.bfloat16)

    def substep(gin, H, C):
        g = gin + _mmb(H, W_hh1b, ((1,), (1,)))
        i = _sig(g[:, :SE])
        f = _sig(g[:, SE:2 * SE])
        gg = jnp.tanh(g[:, 2 * SE:3 * SE])
        o = _sig(g[:, 3 * SE:])
        C = f * C + i * gg
        return o * jnp.tanh(C), C

    H = jnp.zeros((T, SE), _F32)
    C = jnp.zeros((T, SE), _F32)
    gvs = {}
    for s in range(D):
        gin = jnp.concatenate([g[s:s + 1, :] for g in gin_scr], axis=0)
        H, C = substep(gin, H, C)
        if s % 32 == 8:
            gvs[s // 32] = gcn(s // 32)
    H1 = H
    gvs = [gvs[t] for t in range(T)]

    sg = jnp.concatenate([H1, jnp.concatenate(gvs, axis=0)], axis=1)
    sv = jax.nn.relu(_mm(sg, Wsw_ref[...], ((1,), (0,))) + bsw_ref[...])

    W_hh2 = W_hh2_ref[...]
    gin2 = _mm(sv, W_ih2_ref[...], ((1,), (1,))) + bg2_ref[...]
    h = jnp.zeros((1, SE), _F32)
    c = jnp.zeros((1, SE), _F32)
    for s in range(T):
        g = gin2[s:s + 1] + _mm(h, W_hh2, ((1,), (1,)))
        i = _sig(g[:, :SE])
        f = _sig(g[:, SE:2 * SE])
        gg = jnp.tanh(g[:, 2 * SE:3 * SE])
        o = _sig(g[:, 3 * SE:])
        c = f * c + i * gg
        h = o * jnp.tanh(c)

    logits = _mm(h, Wc_ref[...], ((1,), (0,))) + bc_ref[...]
    m = jnp.max(logits, axis=1, keepdims=True)
    z = logits - m
    out_ref[...] = z - jnp.log(jnp.sum(jnp.exp(z), axis=1, keepdims=True))


def _forward(adj_mat_array, node_att_array, W1, b1, W2, b2, W_ih1, W_hh1,
             b_ih1, b_hh1, Wsw, bsw, W_ih2, W_hh2, b_ih2, b_hh2, Wc, bc,
             interpret=False):
    bg1 = (b_ih1 + b_hh1).reshape(1, -1)
    bg2 = (b_ih2 + b_hh2).reshape(1, -1)
    return pl.pallas_call(
        _body,
        out_shape=jax.ShapeDtypeStruct((1, NC), _F32),
        scratch_shapes=[pltpu.VMEM((D, 4 * SE), _F32) for _ in range(T)],
        interpret=interpret,
    )(adj_mat_array, node_att_array, W1, b1.reshape(1, -1), W2,
      b2.reshape(1, -1), W_ih1, W_hh1, bg1, Wsw, bsw.reshape(1, -1),
      W_ih2, W_hh2, bg2, Wc, bc.reshape(1, -1))


def kernel(adj_mat_array, node_att_array, W1, b1, W2, b2, W_ih1, W_hh1,
           b_ih1, b_hh1, Wsw, bsw, W_ih2, W_hh2, b_ih2, b_hh2, Wc, bc):
    return _forward(adj_mat_array, node_att_array, W1, b1, W2, b2, W_ih1,
                    W_hh1, b_ih1, b_hh1, Wsw, bsw, W_ih2, W_hh2, b_ih2,
                    b_hh2, Wc, bc)
